# baseline (speedup 1.0000x reference)
import functools

import jax
import jax.numpy as jnp
from jax.experimental import pallas as pl
from jax.experimental.pallas import tpu as pltpu
from jax.experimental.pallas import tpu_sc as plsc

B = 64
L = 512
H = 512
T = 32
NCHUNK = L // T

N_IDX = L * B
SC_NW = 32
SC_ROWS = N_IDX // SC_NW
SC_CH = 64
SC_NCH = SC_ROWS // SC_CH


def _sc_gather(table, idx):
    mesh = plsc.VectorSubcoreMesh(core_axis_name="c", subcore_axis_name="s")

    @pl.kernel(
        out_type=jax.ShapeDtypeStruct((N_IDX, H), table.dtype),
        mesh=mesh,
        scratch_types=[
            pltpu.VMEM((SC_CH,), jnp.int32),
            pltpu.VMEM((SC_CH,), jnp.int32),
            pltpu.VMEM((SC_CH, H), table.dtype),
            pltpu.VMEM((SC_CH, H), table.dtype),
            pltpu.SemaphoreType.DMA,
            pltpu.SemaphoreType.DMA,
            pltpu.SemaphoreType.DMA,
            pltpu.SemaphoreType.DMA,
        ])
    def k(tab_hbm, i_hbm, o_hbm, idx0, idx1, rows0, rows1,
          sg0, sg1, so0, so1):
        wid = jax.lax.axis_index("s") * 2 + jax.lax.axis_index("c")
        base = wid * SC_ROWS

        def start_gather(c, idx_v, rows_v, sem):
            pltpu.sync_copy(i_hbm.at[pl.ds(base + c * SC_CH, SC_CH)], idx_v)
            return pltpu.async_copy(tab_hbm.at[idx_v], rows_v, sem)

        def out_copy(c, rows_v, sem):
            return pltpu.make_async_copy(
                rows_v, o_hbm.at[pl.ds(base + c * SC_CH, SC_CH)], sem)

        @pl.loop(0, SC_NCH, step=2)
        def _(c):
            @pl.when(c > 0)
            def _():
                out_copy(0, rows0, so0).wait()

            g0 = start_gather(c, idx0, rows0, sg0)

            @pl.when(c > 0)
            def _():
                out_copy(0, rows1, so1).wait()

            g1 = start_gather(c + 1, idx1, rows1, sg1)
            g0.wait()
            out_copy(c, rows0, so0).start()
            g1.wait()
            out_copy(c + 1, rows1, so1).start()

        out_copy(0, rows0, so0).wait()
        out_copy(0, rows1, so1).wait()

    return k(table, idx)


def _gru_steps(i, t0, n_steps, lengths, whh, bhh, xp_scr, h, o_scr):
    def step(t, h):
        hp_rz = jnp.dot(h, whh[:, :2 * H], preferred_element_type=jnp.float32)
        hp_n = jnp.dot(h, whh[:, 2 * H:], preferred_element_type=jnp.float32)
        xp_t = xp_scr[pl.ds(t * B, B), :]
        g = xp_t[:, :2 * H] + hp_rz.astype(jnp.bfloat16)
        r = jax.nn.sigmoid(g[:, :H])
        z = jax.nn.sigmoid(g[:, H:])
        n = jnp.tanh(xp_t[:, 2 * H:]
                     + r * (hp_n + bhh).astype(jnp.bfloat16))
        h_new = n + z * (h - n)
        m = (i * T + t) < lengths
        if o_scr is not None:
            o_scr[pl.ds(t * B, B), :] = jnp.where(
                m, h_new, jnp.bfloat16(0.0))
        return jnp.where(m, h_new, h)

    return jax.lax.fori_loop(t0, t0 + n_steps, step, h, unroll=4)


def _enc_kernel(len_ref, emb_ref, wih_ref, whh_ref, bih_ref, bhh_ref,
                hout_ref, xp_scr, h_scr):
    i = pl.program_id(0)

    @pl.when(i == 0)
    def _():
        h_scr[...] = jnp.zeros((B, H), jnp.float32)

    xp_scr[...] = (
        jnp.dot(emb_ref[...].astype(jnp.bfloat16), wih_ref[...],
                preferred_element_type=jnp.float32)
        + bih_ref[...]).astype(jnp.bfloat16)

    h = _gru_steps(i, 0, T, len_ref[...], whh_ref[...], bhh_ref[...],
                   xp_scr, h_scr[...].astype(jnp.bfloat16), None)
    hf = h.astype(jnp.float32)
    h_scr[...] = hf
    hout_ref[...] = hf


def _dec_kernel(len_ref, emb_ref, wih_ref, whh_ref, bih_ref, bhh_ref,
                h0_ref, fcw_ref, fcb_ref, out_ref, xp_scr, o_scr, h_scr):
    i = pl.program_id(0)

    @pl.when(i == 0)
    def _():
        h_scr[...] = h0_ref[...]

    xp_scr[...] = (
        jnp.dot(emb_ref[...].astype(jnp.bfloat16), wih_ref[...],
                preferred_element_type=jnp.float32)
        + bih_ref[...]).astype(jnp.bfloat16)

    h = _gru_steps(i, 0, T, len_ref[...], whh_ref[...], bhh_ref[...],
                   xp_scr, h_scr[...].astype(jnp.bfloat16), o_scr)
    h_scr[...] = h.astype(jnp.float32)

    logits = (
        jnp.dot(o_scr[...], fcw_ref[...],
                preferred_element_type=jnp.float32)
        + fcb_ref[...])
    mx = jnp.max(logits, axis=-1, keepdims=True)
    lse = jnp.log(jnp.sum(jnp.exp(logits - mx), axis=-1, keepdims=True)) + mx
    out = logits - lse
    out_ref[...] = jnp.swapaxes(out.reshape(T, B, H), 0, 1)


def _const_spec(shape):
    return pl.BlockSpec(shape, lambda i: tuple(0 for _ in shape))


def _run_encoder(lengths, emb, wih_t, whh_t, bih, bhh, interpret=False):
    return pl.pallas_call(
        _enc_kernel,
        grid=(NCHUNK,),
        in_specs=[
            _const_spec((B, 1)),
            pl.BlockSpec((T * B, H), lambda i: (i, 0)),
            _const_spec((H, 3 * H)),
            _const_spec((H, 3 * H)),
            _const_spec((1, 3 * H)),
            _const_spec((1, H)),
        ],
        out_specs=_const_spec((B, H)),
        out_shape=jax.ShapeDtypeStruct((B, H), jnp.float32),
        scratch_shapes=[
            pltpu.VMEM((T * B, 3 * H), jnp.bfloat16),
            pltpu.VMEM((B, H), jnp.float32),
        ],
        interpret=interpret,
    )(lengths, emb, wih_t, whh_t, bih, bhh)


def _run_decoder(lengths, emb, wih_t, whh_t, bih, bhh, h0, fcw_t, fcb,
                 interpret=False):
    return pl.pallas_call(
        _dec_kernel,
        grid=(NCHUNK,),
        in_specs=[
            _const_spec((B, 1)),
            pl.BlockSpec((T * B, H), lambda i: (i, 0)),
            _const_spec((H, 3 * H)),
            _const_spec((H, 3 * H)),
            _const_spec((1, 3 * H)),
            _const_spec((1, H)),
            _const_spec((B, H)),
            _const_spec((H, H)),
            _const_spec((1, H)),
        ],
        out_specs=pl.BlockSpec((B, T, H), lambda i: (0, i, 0)),
        out_shape=jax.ShapeDtypeStruct((B, L, H), jnp.float32),
        scratch_shapes=[
            pltpu.VMEM((T * B, 3 * H), jnp.bfloat16),
            pltpu.VMEM((T * B, H), jnp.bfloat16),
            pltpu.VMEM((B, H), jnp.float32),
        ],
        interpret=interpret,
    )(lengths, emb, wih_t, whh_t, bih, bhh, h0, fcw_t, fcb)


def kernel(x, x_lengths, y, y_lengths, embed_en, W_ih_e, W_hh_e, b_ih_e,
           b_hh_e, embed_cn, W_ih_d, W_hh_d, b_ih_d, b_hh_d, fc_W, fc_b,
           interpret=False):
    enc_emb = _sc_gather(embed_en, x.T.reshape(-1))
    dec_emb = _sc_gather(embed_cn, y.T.reshape(-1))

    xlen = x_lengths.astype(jnp.int32).reshape(B, 1)
    ylen = y_lengths.astype(jnp.int32).reshape(B, 1)

    bf = jnp.bfloat16
    zH = jnp.zeros((H,), jnp.float32)
    bx_e = (b_ih_e + jnp.concatenate([b_hh_e[:2 * H], zH])).reshape(1, -1)
    bx_d = (b_ih_d + jnp.concatenate([b_hh_d[:2 * H], zH])).reshape(1, -1)
    enc_h = _run_encoder(xlen, enc_emb, W_ih_e.T.astype(bf),
                         W_hh_e.T.astype(bf),
                         bx_e, b_hh_e[2 * H:].reshape(1, -1),
                         interpret=interpret)
    out2d = _run_decoder(ylen, dec_emb, W_ih_d.T.astype(bf),
                         W_hh_d.T.astype(bf),
                         bx_d, b_hh_d[2 * H:].reshape(1, -1),
                         enc_h, fc_W.T.astype(bf), fc_b.reshape(1, -1),
                         interpret=interpret)
    return out2d


# speedup vs baseline: 1.0176x; 1.0176x over previous
import functools

import jax
import jax.numpy as jnp
from jax.experimental import pallas as pl
from jax.experimental.pallas import tpu as pltpu
from jax.experimental.pallas import tpu_sc as plsc

B = 64
L = 512
H = 512
T = 32
NCHUNK = L // T
TE = 64
NCHUNK_E = L // TE

N_IDX = L * B
SC_NW = 32
SC_ROWS = N_IDX // SC_NW
SC_CH = 64
SC_NCH = SC_ROWS // SC_CH


def _sc_gather(table, idx):
    mesh = plsc.VectorSubcoreMesh(core_axis_name="c", subcore_axis_name="s")

    @pl.kernel(
        out_type=jax.ShapeDtypeStruct((N_IDX, H), table.dtype),
        mesh=mesh,
        scratch_types=[
            pltpu.VMEM((SC_CH,), jnp.int32),
            pltpu.VMEM((SC_CH,), jnp.int32),
            pltpu.VMEM((SC_CH, H), table.dtype),
            pltpu.VMEM((SC_CH, H), table.dtype),
            pltpu.SemaphoreType.DMA,
            pltpu.SemaphoreType.DMA,
            pltpu.SemaphoreType.DMA,
            pltpu.SemaphoreType.DMA,
        ])
    def k(tab_hbm, i_hbm, o_hbm, idx0, idx1, rows0, rows1,
          sg0, sg1, so0, so1):
        wid = jax.lax.axis_index("s") * 2 + jax.lax.axis_index("c")
        base = wid * SC_ROWS

        def start_gather(c, idx_v, rows_v, sem):
            pltpu.sync_copy(i_hbm.at[pl.ds(base + c * SC_CH, SC_CH)], idx_v)
            return pltpu.async_copy(tab_hbm.at[idx_v], rows_v, sem)

        def out_copy(c, rows_v, sem):
            return pltpu.make_async_copy(
                rows_v, o_hbm.at[pl.ds(base + c * SC_CH, SC_CH)], sem)

        @pl.loop(0, SC_NCH, step=2)
        def _(c):
            @pl.when(c > 0)
            def _():
                out_copy(0, rows0, so0).wait()

            g0 = start_gather(c, idx0, rows0, sg0)

            @pl.when(c > 0)
            def _():
                out_copy(0, rows1, so1).wait()

            g1 = start_gather(c + 1, idx1, rows1, sg1)
            g0.wait()
            out_copy(c, rows0, so0).start()
            g1.wait()
            out_copy(c + 1, rows1, so1).start()

        out_copy(0, rows0, so0).wait()
        out_copy(0, rows1, so1).wait()

    return k(table, idx)


def _gru_steps(i, t0, n_steps, chunk, lengths, whh, bhh, xp_scr, h, o_scr):
    def step(t, h):
        hp_rz = jnp.dot(h, whh[:, :2 * H], preferred_element_type=jnp.float32)
        hp_n = jnp.dot(h, whh[:, 2 * H:], preferred_element_type=jnp.float32)
        xp_t = xp_scr[pl.ds(t * B, B), :]
        g = xp_t[:, :2 * H] + hp_rz.astype(jnp.bfloat16)
        r = jax.nn.sigmoid(g[:, :H])
        z = jax.nn.sigmoid(g[:, H:])
        n = jnp.tanh(xp_t[:, 2 * H:]
                     + r * (hp_n + bhh).astype(jnp.bfloat16))
        h_new = n + z * (h - n)
        m = (i * chunk + t) < lengths
        if o_scr is not None:
            o_scr[pl.ds(t * B, B), :] = jnp.where(
                m, h_new, jnp.bfloat16(0.0))
        return jnp.where(m, h_new, h)

    return jax.lax.fori_loop(t0, t0 + n_steps, step, h, unroll=4)


def _enc_kernel(len_ref, emb_ref, wih_ref, whh_ref, bih_ref, bhh_ref,
                hout_ref, xp_scr, h_scr):
    i = pl.program_id(0)

    @pl.when(i == 0)
    def _():
        h_scr[...] = jnp.zeros((B, H), jnp.float32)

    xp_scr[...] = (
        jnp.dot(emb_ref[...].astype(jnp.bfloat16), wih_ref[...],
                preferred_element_type=jnp.float32)
        + bih_ref[...]).astype(jnp.bfloat16)

    h = _gru_steps(i, 0, TE, TE, len_ref[...], whh_ref[...], bhh_ref[...],
                   xp_scr, h_scr[...].astype(jnp.bfloat16), None)
    hf = h.astype(jnp.float32)
    h_scr[...] = hf
    hout_ref[...] = hf


def _dec_kernel(len_ref, emb_ref, wih_ref, whh_ref, bih_ref, bhh_ref,
                h0_ref, fcw_ref, fcb_ref, out_ref, xp_scr, o_scr, h_scr):
    i = pl.program_id(0)

    @pl.when(i == 0)
    def _():
        h_scr[...] = h0_ref[...]

    xp_scr[...] = (
        jnp.dot(emb_ref[...].astype(jnp.bfloat16), wih_ref[...],
                preferred_element_type=jnp.float32)
        + bih_ref[...]).astype(jnp.bfloat16)

    h = _gru_steps(i, 0, T, T, len_ref[...], whh_ref[...], bhh_ref[...],
                   xp_scr, h_scr[...].astype(jnp.bfloat16), o_scr)
    h_scr[...] = h.astype(jnp.float32)

    logits = (
        jnp.dot(o_scr[...], fcw_ref[...],
                preferred_element_type=jnp.float32)
        + fcb_ref[...])
    mx = jnp.max(logits, axis=-1, keepdims=True)
    lse = jnp.log(jnp.sum(jnp.exp(logits - mx), axis=-1, keepdims=True)) + mx
    out = logits - lse
    out_ref[...] = jnp.swapaxes(out.reshape(T, B, H), 0, 1)


def _const_spec(shape):
    return pl.BlockSpec(shape, lambda i: tuple(0 for _ in shape))


def _run_encoder(lengths, emb, wih_t, whh_t, bih, bhh, interpret=False):
    return pl.pallas_call(
        _enc_kernel,
        grid=(NCHUNK_E,),
        in_specs=[
            _const_spec((B, 1)),
            pl.BlockSpec((TE * B, H), lambda i: (i, 0)),
            _const_spec((H, 3 * H)),
            _const_spec((H, 3 * H)),
            _const_spec((1, 3 * H)),
            _const_spec((1, H)),
        ],
        out_specs=_const_spec((B, H)),
        out_shape=jax.ShapeDtypeStruct((B, H), jnp.float32),
        scratch_shapes=[
            pltpu.VMEM((TE * B, 3 * H), jnp.bfloat16),
            pltpu.VMEM((B, H), jnp.float32),
        ],
        interpret=interpret,
    )(lengths, emb, wih_t, whh_t, bih, bhh)


def _run_decoder(lengths, emb, wih_t, whh_t, bih, bhh, h0, fcw_t, fcb,
                 interpret=False):
    return pl.pallas_call(
        _dec_kernel,
        grid=(NCHUNK,),
        in_specs=[
            _const_spec((B, 1)),
            pl.BlockSpec((T * B, H), lambda i: (i, 0)),
            _const_spec((H, 3 * H)),
            _const_spec((H, 3 * H)),
            _const_spec((1, 3 * H)),
            _const_spec((1, H)),
            _const_spec((B, H)),
            _const_spec((H, H)),
            _const_spec((1, H)),
        ],
        out_specs=pl.BlockSpec((B, T, H), lambda i: (0, i, 0)),
        out_shape=jax.ShapeDtypeStruct((B, L, H), jnp.float32),
        scratch_shapes=[
            pltpu.VMEM((T * B, 3 * H), jnp.bfloat16),
            pltpu.VMEM((T * B, H), jnp.bfloat16),
            pltpu.VMEM((B, H), jnp.float32),
        ],
        interpret=interpret,
    )(lengths, emb, wih_t, whh_t, bih, bhh, h0, fcw_t, fcb)


def kernel(x, x_lengths, y, y_lengths, embed_en, W_ih_e, W_hh_e, b_ih_e,
           b_hh_e, embed_cn, W_ih_d, W_hh_d, b_ih_d, b_hh_d, fc_W, fc_b,
           interpret=False):
    enc_emb = _sc_gather(embed_en, x.T.reshape(-1))
    dec_emb = _sc_gather(embed_cn, y.T.reshape(-1))

    xlen = x_lengths.astype(jnp.int32).reshape(B, 1)
    ylen = y_lengths.astype(jnp.int32).reshape(B, 1)

    bf = jnp.bfloat16
    zH = jnp.zeros((H,), jnp.float32)
    bx_e = (b_ih_e + jnp.concatenate([b_hh_e[:2 * H], zH])).reshape(1, -1)
    bx_d = (b_ih_d + jnp.concatenate([b_hh_d[:2 * H], zH])).reshape(1, -1)
    enc_h = _run_encoder(xlen, enc_emb, W_ih_e.T.astype(bf),
                         W_hh_e.T.astype(bf),
                         bx_e, b_hh_e[2 * H:].reshape(1, -1),
                         interpret=interpret)
    out2d = _run_decoder(ylen, dec_emb, W_ih_d.T.astype(bf),
                         W_hh_d.T.astype(bf),
                         bx_d, b_hh_d[2 * H:].reshape(1, -1),
                         enc_h, fc_W.T.astype(bf), fc_b.reshape(1, -1),
                         interpret=interpret)
    return out2d
